# baseline (speedup 1.0000x reference)
import functools

import jax
import jax.numpy as jnp
from jax import lax
from jax.experimental import pallas as pl
from jax.experimental.pallas import tpu as pltpu
from jax.experimental.pallas import tpu_sc as plsc

NC = 2
NS = 16
L = 16
NW = NC * NS

CW = 512


def _sc_mesh():
    return plsc.VectorSubcoreMesh(
        core_axis_name="c", subcore_axis_name="s",
        num_cores=NC, num_subcores=NS)


def _stage_row_major(t2, tail_flat, V, D):
    n_chunk = V // CW
    tail0 = n_chunk * CW
    tailw = V - tail0
    n_pair = (n_chunk // NW + 1) // 2

    @functools.partial(
        pl.kernel,
        out_type=jax.ShapeDtypeStruct((V * D,), jnp.float32),
        mesh=_sc_mesh(),
        compiler_params=pltpu.CompilerParams(
            needs_layout_passes=False, use_tc_tiling_on_sc=True),
        scratch_types=[
            pltpu.VMEM((D, CW), jnp.float32),
            pltpu.VMEM((D, CW), jnp.float32),
            pltpu.VMEM((CW * D,), jnp.float32),
            pltpu.VMEM((CW * D,), jnp.float32),
            pltpu.SemaphoreType.DMA,
            pltpu.SemaphoreType.DMA,
            pltpu.SemaphoreType.DMA,
            pltpu.SemaphoreType.DMA,
        ],
    )
    def stage(t2_h, tail_h, out_h, t_a, t_b, r_a, r_b,
              sem_a, sem_b, so_a, so_b):
        wid = lax.axis_index("s") * NC + lax.axis_index("c")
        lanes = lax.iota(jnp.int32, L)
        row0 = lanes
        row1 = lanes + L

        def fire(c, tbuf, sem):
            pltpu.async_copy(t2_h.at[:, pl.ds(c * CW, CW)], tbuf, sem)

        def drain(c, tbuf, sem):
            pltpu.make_async_copy(t2_h.at[:, pl.ds(c * CW, CW)],
                                  tbuf, sem).wait()

        perm = [jnp.bitwise_and(lanes + j, L - 1) for j in range(L)]
        svec = [perm[j] * D + lanes for j in range(L)]
        rows_h = [row0, row1]

        def transpose(tbuf, rbuf):
            def tr(vb, _):
                gs = []
                for hb in range(D // L):
                    for j in range(L):
                        colv = perm[j] + L * vb
                        gs.append(plsc.load_gather(tbuf, [rows_h[hb], colv]))
                for hb in range(D // L):
                    for j in range(L):
                        sidx = svec[j] + (L * D * vb + L * hb)
                        plsc.store_scatter(rbuf, [sidx],
                                           gs[hb * L + j])
                return 0
            lax.fori_loop(0, CW // L, tr, 0)

        def fire_out(c, rbuf, so):
            pltpu.async_copy(rbuf, out_h.at[pl.ds(c * CW * D, CW * D)], so)

        def wait_out(rbuf, so):
            pltpu.make_async_copy(rbuf, out_h.at[pl.ds(0, CW * D)],
                                  so).wait()

        fire(wid, t_a, sem_a)

        def body(t, _):
            c_a = wid + 2 * NW * t
            c_b = c_a + NW
            c_c = c_a + 2 * NW
            drain(c_a, t_a, sem_a)

            @pl.when(c_b < n_chunk)
            def _():
                fire(c_b, t_b, sem_b)

            @pl.when(t > 0)
            def _():
                wait_out(r_a, so_a)

            transpose(t_a, r_a)
            fire_out(c_a, r_a, so_a)

            @pl.when(c_b < n_chunk)
            def _():
                drain(c_b, t_b, sem_b)

                @pl.when(c_c < n_chunk)
                def _():
                    fire(c_c, t_a, sem_a)

                @pl.when(t > 0)
                def _():
                    wait_out(r_b, so_b)

                transpose(t_b, r_b)
                fire_out(c_b, r_b, so_b)

            return 0

        lax.fori_loop(0, n_pair, body, 0)
        wait_out(r_a, so_a)
        wait_out(r_b, so_b)

        if tailw:
            @pl.when(wid == NW - 1)
            def _():
                pltpu.sync_copy(tail_h, r_a.at[pl.ds(0, tailw * D)])
                pltpu.sync_copy(r_a.at[pl.ds(0, tailw * D)],
                                out_h.at[pl.ds(tail0 * D, tailw * D)])

    return stage(t2, tail_flat)


SB = 1


def _gather_pool_project(text, table2, W, b, S, B, V, D, O):
    bpw = B // NW
    assert S % (2 * SB) == 0

    @functools.partial(
        pl.kernel,
        out_type=jax.ShapeDtypeStruct((B * O,), jnp.float32),
        mesh=_sc_mesh(),
        compiler_params=pltpu.CompilerParams(
            needs_layout_passes=False, use_tc_tiling_on_sc=False),
        scratch_types=[
            pltpu.VMEM((S * bpw,), jnp.int32),
            pltpu.VMEM((SB * bpw, D), jnp.float32),
            pltpu.VMEM((SB * bpw, D), jnp.float32),
            pltpu.VMEM((SB * bpw, D), jnp.float32),
            pltpu.VMEM((SB * bpw, D), jnp.float32),
            pltpu.VMEM((bpw, D), jnp.float32),
            pltpu.SemaphoreType.DMA,
            pltpu.SemaphoreType.DMA,
            pltpu.SemaphoreType.DMA,
            pltpu.SemaphoreType.DMA,
            pltpu.VMEM((O, D), jnp.float32),
            pltpu.VMEM((L,), jnp.float32),
            pltpu.VMEM((bpw * D,), jnp.float32),
            pltpu.VMEM((bpw * O,), jnp.float32),
        ],
    )
    def fasttext_sc(text_h, table_h, w_h, b_h, out_h,
                    idx_v, rows0, rows1, rows2, rows3, acc_v,
                    sem0, sem1, sem2, sem3,
                    w_v, b_v, flat_v, out_v):
        wid = lax.axis_index("s") * NC + lax.axis_index("c")
        base = wid * bpw

        for s in range(S):
            pltpu.async_copy(text_h.at[s, pl.ds(base, bpw)],
                             idx_v.at[pl.ds(s * bpw, bpw)], sem0)
        for s in range(S):
            pltpu.make_async_copy(text_h.at[s, pl.ds(base, bpw)],
                                  idx_v.at[pl.ds(s * bpw, bpw)], sem0).wait()
        pltpu.sync_copy(w_h, w_v)
        pltpu.sync_copy(b_h, b_v.at[pl.ds(0, O)])

        zero = jnp.zeros((L,), jnp.float32)
        for r in range(bpw):
            for h in range(D // L):
                acc_v[r, pl.ds(h * L, L)] = zero

        def accumulate(buf):
            for si in range(SB):
                for r in range(bpw):
                    for h in range(D // L):
                        plsc.addupdate(
                            acc_v.at[r, pl.ds(h * L, L)],
                            buf[si * bpw + r, pl.ds(h * L, L)])

        nb = SB * bpw

        def gather_start(g, buf, sem):
            pltpu.async_copy(
                table_h.at[idx_v.at[pl.ds(g * nb, nb)]], buf, sem)

        def gather_wait(g, buf, sem):
            pltpu.make_async_copy(
                table_h.at[idx_v.at[pl.ds(g * nb, nb)]], buf, sem).wait()

        ng = S // SB
        ring = [(rows0, sem0), (rows1, sem1), (rows2, sem2), (rows3, sem3)]
        nr = len(ring)
        for k in range(nr):
            gather_start(k, *ring[k])

        def seq_block(t, _):
            g0 = nr * t
            for k in range(nr):
                buf, sem = ring[k]
                gather_wait(g0 + k, buf, sem)
                accumulate(buf)

                @pl.when(g0 + k + nr < ng)
                def _():
                    gather_start(g0 + k + nr, buf, sem)

            return 0

        lax.fori_loop(0, ng // nr, seq_block, 0)

        for r in range(bpw):
            for h in range(D // L):
                flat_v[pl.ds(r * D + h * L, L)] = acc_v[r, pl.ds(h * L, L)]

        inv_s = jnp.float32(1.0 / S)
        lanes = lax.iota(jnp.int32, L)
        w_rows = [[w_v[o, pl.ds(h * L, L)] for h in range(D // L)]
                  for o in range(O)]
        ws = [[w_rows[o][d // L][d % L] for d in range(D)] for o in range(O)]
        b_vec = b_v[pl.ds(0, L)]
        bs = [b_vec[o] for o in range(O)]
        for g in range(bpw // L):
            row_idx = (g * L + lanes) * D
            outs = [jnp.zeros((L,), jnp.float32) for _ in range(O)]
            for d in range(D):
                vals = plsc.load_gather(flat_v, [row_idx + d])
                for o in range(O):
                    outs[o] = outs[o] + vals * ws[o][d]
            for o in range(O):
                res = outs[o] * inv_s + bs[o]
                plsc.store_scatter(out_v, [(g * L + lanes) * O + o], res)

        pltpu.sync_copy(out_v, out_h.at[pl.ds(base * O, bpw * O)])

    return fasttext_sc(text, table2, W, b)


@jax.jit
def kernel(text, table, W, b):
    S, B = text.shape
    V, D = table.shape
    O = W.shape[0]
    assert B % NW == 0 and D == 2 * L

    tail0 = (V // CW) * CW
    tail_flat = table[tail0:].reshape(-1)
    flat = _stage_row_major(table.T, tail_flat, V, D)
    table2 = flat.reshape(V, D)
    out = _gather_pool_project(text, table2, W, b, S, B, V, D, O)
    return out.reshape(B, O)


# speedup vs baseline: 1.0518x; 1.0518x over previous
import functools

import jax
import jax.numpy as jnp
from jax import lax
from jax.experimental import pallas as pl
from jax.experimental.pallas import tpu as pltpu
from jax.experimental.pallas import tpu_sc as plsc

NC = 2
NS = 16
L = 16
NW = NC * NS

CW = 512


def _sc_mesh():
    return plsc.VectorSubcoreMesh(
        core_axis_name="c", subcore_axis_name="s",
        num_cores=NC, num_subcores=NS)


def _stage_row_major(t2, tail_flat, V, D):
    n_chunk = V // CW
    tail0 = n_chunk * CW
    tailw = V - tail0
    n_pair = (n_chunk // NW + 1) // 2

    @functools.partial(
        pl.kernel,
        out_type=jax.ShapeDtypeStruct((V * D,), jnp.float32),
        mesh=_sc_mesh(),
        compiler_params=pltpu.CompilerParams(
            needs_layout_passes=False, use_tc_tiling_on_sc=True),
        scratch_types=[
            pltpu.VMEM((D, CW), jnp.float32),
            pltpu.VMEM((D, CW), jnp.float32),
            pltpu.VMEM((CW * D,), jnp.float32),
            pltpu.VMEM((CW * D,), jnp.float32),
            pltpu.SemaphoreType.DMA,
            pltpu.SemaphoreType.DMA,
            pltpu.SemaphoreType.DMA,
            pltpu.SemaphoreType.DMA,
        ],
    )
    def stage(t2_h, tail_h, out_h, t_a, t_b, r_a, r_b,
              sem_a, sem_b, so_a, so_b):
        wid = lax.axis_index("s") * NC + lax.axis_index("c")
        lanes = lax.iota(jnp.int32, L)
        row0 = lanes
        row1 = lanes + L

        def fire(c, tbuf, sem):
            pltpu.async_copy(t2_h.at[:, pl.ds(c * CW, CW)], tbuf, sem)

        def drain(c, tbuf, sem):
            pltpu.make_async_copy(t2_h.at[:, pl.ds(c * CW, CW)],
                                  tbuf, sem).wait()

        perm = [jnp.bitwise_and(lanes + j, L - 1) for j in range(L)]
        svec = [perm[j] * D + lanes for j in range(L)]
        rows_h = [row0, row1]

        def transpose(tbuf, rbuf):
            def tr(vb, _):
                gs = []
                for hb in range(D // L):
                    for j in range(L):
                        colv = perm[j] + L * vb
                        gs.append(plsc.load_gather(tbuf, [rows_h[hb], colv]))
                for hb in range(D // L):
                    for j in range(L):
                        sidx = svec[j] + (L * D * vb + L * hb)
                        plsc.store_scatter(rbuf, [sidx],
                                           gs[hb * L + j])
                return 0
            lax.fori_loop(0, CW // L, tr, 0)

        def fire_out(c, rbuf, so):
            pltpu.async_copy(rbuf, out_h.at[pl.ds(c * CW * D, CW * D)], so)

        def wait_out(rbuf, so):
            pltpu.make_async_copy(rbuf, out_h.at[pl.ds(0, CW * D)],
                                  so).wait()

        fire(wid, t_a, sem_a)

        def body(t, _):
            c_a = wid + 2 * NW * t
            c_b = c_a + NW
            c_c = c_a + 2 * NW
            drain(c_a, t_a, sem_a)

            @pl.when(c_b < n_chunk)
            def _():
                fire(c_b, t_b, sem_b)

            @pl.when(t > 0)
            def _():
                wait_out(r_a, so_a)

            transpose(t_a, r_a)
            fire_out(c_a, r_a, so_a)

            @pl.when(c_b < n_chunk)
            def _():
                drain(c_b, t_b, sem_b)

                @pl.when(c_c < n_chunk)
                def _():
                    fire(c_c, t_a, sem_a)

                @pl.when(t > 0)
                def _():
                    wait_out(r_b, so_b)

                transpose(t_b, r_b)
                fire_out(c_b, r_b, so_b)

            return 0

        lax.fori_loop(0, n_pair, body, 0)
        wait_out(r_a, so_a)
        wait_out(r_b, so_b)

        if tailw:
            @pl.when(wid == NW - 1)
            def _():
                pltpu.sync_copy(tail_h, r_a.at[pl.ds(0, tailw * D)])
                pltpu.sync_copy(r_a.at[pl.ds(0, tailw * D)],
                                out_h.at[pl.ds(tail0 * D, tailw * D)])

    return stage(t2, tail_flat)


SB = 1


def _gather_pool_project(text, table2, W, b, S, B, V, D, O):
    bpw = B // NW
    assert S % (2 * SB) == 0

    @functools.partial(
        pl.kernel,
        out_type=jax.ShapeDtypeStruct((B * O,), jnp.float32),
        mesh=_sc_mesh(),
        compiler_params=pltpu.CompilerParams(
            needs_layout_passes=False, use_tc_tiling_on_sc=False),
        scratch_types=[
            pltpu.VMEM((S * bpw,), jnp.int32),
            pltpu.VMEM((SB * bpw, D), jnp.float32),
            pltpu.VMEM((SB * bpw, D), jnp.float32),
            pltpu.VMEM((SB * bpw, D), jnp.float32),
            pltpu.VMEM((SB * bpw, D), jnp.float32),
            pltpu.VMEM((bpw, D), jnp.float32),
            pltpu.SemaphoreType.DMA,
            pltpu.SemaphoreType.DMA,
            pltpu.SemaphoreType.DMA,
            pltpu.SemaphoreType.DMA,
            pltpu.VMEM((O, D), jnp.float32),
            pltpu.VMEM((L,), jnp.float32),
            pltpu.VMEM((bpw * D,), jnp.float32),
            pltpu.VMEM((bpw * O,), jnp.float32),
        ],
    )
    def fasttext_sc(text_h, table_h, w_h, b_h, out_h,
                    idx_v, rows0, rows1, rows2, rows3, acc_v,
                    sem0, sem1, sem2, sem3,
                    w_v, b_v, flat_v, out_v):
        wid = lax.axis_index("s") * NC + lax.axis_index("c")
        base = wid * bpw

        for s in range(S):
            pltpu.async_copy(text_h.at[s, pl.ds(base, bpw)],
                             idx_v.at[pl.ds(s * bpw, bpw)], sem0)
        for s in range(S):
            pltpu.make_async_copy(text_h.at[s, pl.ds(base, bpw)],
                                  idx_v.at[pl.ds(s * bpw, bpw)], sem0).wait()
        pltpu.sync_copy(w_h, w_v)
        pltpu.sync_copy(b_h, b_v.at[pl.ds(0, O)])

        zero = jnp.zeros((L,), jnp.float32)
        for r in range(bpw):
            for h in range(D // L):
                acc_v[r, pl.ds(h * L, L)] = zero

        def accumulate(buf):
            for si in range(SB):
                for r in range(bpw):
                    for h in range(D // L):
                        plsc.addupdate(
                            acc_v.at[r, pl.ds(h * L, L)],
                            buf[si * bpw + r, pl.ds(h * L, L)])

        nb = SB * bpw
        hb = nb // 2

        def gather_start(g, buf, sem):
            pltpu.async_copy(
                table_h.at[idx_v.at[pl.ds(g * nb, hb)]],
                buf.at[pl.ds(0, hb), :], sem)
            pltpu.async_copy(
                table_h.at[idx_v.at[pl.ds(g * nb + hb, hb)]],
                buf.at[pl.ds(hb, hb), :], sem)

        def gather_wait(g, buf, sem):
            pltpu.make_async_copy(
                table_h.at[idx_v.at[pl.ds(g * nb, hb)]],
                buf.at[pl.ds(0, hb), :], sem).wait()
            pltpu.make_async_copy(
                table_h.at[idx_v.at[pl.ds(g * nb + hb, hb)]],
                buf.at[pl.ds(hb, hb), :], sem).wait()

        ng = S // SB
        gather_start(0, rows0, sem0)

        def seq_pair(t, _):
            g = 2 * t
            gather_wait(g, rows0, sem0)
            gather_start(g + 1, rows1, sem1)
            accumulate(rows0)
            gather_wait(g + 1, rows1, sem1)

            @pl.when(g + 2 < ng)
            def _():
                gather_start(g + 2, rows0, sem0)

            accumulate(rows1)
            return 0

        lax.fori_loop(0, ng // 2, seq_pair, 0)

        for r in range(bpw):
            for h in range(D // L):
                flat_v[pl.ds(r * D + h * L, L)] = acc_v[r, pl.ds(h * L, L)]

        inv_s = jnp.float32(1.0 / S)
        lanes = lax.iota(jnp.int32, L)
        w_rows = [[w_v[o, pl.ds(h * L, L)] for h in range(D // L)]
                  for o in range(O)]
        ws = [[w_rows[o][d // L][d % L] for d in range(D)] for o in range(O)]
        b_vec = b_v[pl.ds(0, L)]
        bs = [b_vec[o] for o in range(O)]
        for g in range(bpw // L):
            row_idx = (g * L + lanes) * D
            outs = [jnp.zeros((L,), jnp.float32) for _ in range(O)]
            for d in range(D):
                vals = plsc.load_gather(flat_v, [row_idx + d])
                for o in range(O):
                    outs[o] = outs[o] + vals * ws[o][d]
            for o in range(O):
                res = outs[o] * inv_s + bs[o]
                plsc.store_scatter(out_v, [(g * L + lanes) * O + o], res)

        pltpu.sync_copy(out_v, out_h.at[pl.ds(base * O, bpw * O)])

    return fasttext_sc(text, table2, W, b)


@jax.jit
def kernel(text, table, W, b):
    S, B = text.shape
    V, D = table.shape
    O = W.shape[0]
    assert B % NW == 0 and D == 2 * L

    tail0 = (V // CW) * CW
    tail_flat = table[tail0:].reshape(-1)
    flat = _stage_row_major(table.T, tail_flat, V, D)
    table2 = flat.reshape(V, D)
    out = _gather_pool_project(text, table2, W, b, S, B, V, D, O)
    return out.reshape(B, O)


# speedup vs baseline: 1.4144x; 1.3447x over previous
import functools

import jax
import jax.numpy as jnp
from jax import lax
from jax.experimental import pallas as pl
from jax.experimental.pallas import tpu as pltpu
from jax.experimental.pallas import tpu_sc as plsc

NC = 2
NS = 16
L = 16
NW = NC * NS

CW = 512


def _sc_mesh():
    return plsc.VectorSubcoreMesh(
        core_axis_name="c", subcore_axis_name="s",
        num_cores=NC, num_subcores=NS)


def _stage_row_major(t2, tail_flat, V, D):
    n_chunk = V // CW
    tail0 = n_chunk * CW
    tailw = V - tail0
    n_pair = (n_chunk // NW + 1) // 2

    @functools.partial(
        pl.kernel,
        out_type=jax.ShapeDtypeStruct((V * D,), jnp.float32),
        mesh=_sc_mesh(),
        compiler_params=pltpu.CompilerParams(
            needs_layout_passes=False, use_tc_tiling_on_sc=True),
        scratch_types=[
            pltpu.VMEM((D, CW), jnp.float32),
            pltpu.VMEM((D, CW), jnp.float32),
            pltpu.VMEM((CW * D,), jnp.float32),
            pltpu.VMEM((CW * D,), jnp.float32),
            pltpu.SemaphoreType.DMA,
            pltpu.SemaphoreType.DMA,
            pltpu.SemaphoreType.DMA,
            pltpu.SemaphoreType.DMA,
        ],
    )
    def stage(t2_h, tail_h, out_h, t_a, t_b, r_a, r_b,
              sem_a, sem_b, so_a, so_b):
        wid = lax.axis_index("s") * NC + lax.axis_index("c")
        lanes = lax.iota(jnp.int32, L)
        row0 = lanes
        row1 = lanes + L

        def fire(c, tbuf, sem):
            pltpu.async_copy(t2_h.at[:, pl.ds(c * CW, CW)], tbuf, sem)

        def drain(c, tbuf, sem):
            pltpu.make_async_copy(t2_h.at[:, pl.ds(c * CW, CW)],
                                  tbuf, sem).wait()

        perm = [jnp.bitwise_and(lanes + j, L - 1) for j in range(L)]
        svec = [perm[j] * D + lanes for j in range(L)]
        rows_h = [row0, row1]

        def transpose(tbuf, rbuf):
            def tr(vb, _):
                gs = []
                for hb in range(D // L):
                    for j in range(L):
                        colv = perm[j] + L * vb
                        gs.append(plsc.load_gather(tbuf, [rows_h[hb], colv]))
                for hb in range(D // L):
                    for j in range(L):
                        sidx = svec[j] + (L * D * vb + L * hb)
                        plsc.store_scatter(rbuf, [sidx],
                                           gs[hb * L + j])
                return 0
            lax.fori_loop(0, CW // L, tr, 0)

        def fire_out(c, rbuf, so):
            pltpu.async_copy(rbuf, out_h.at[pl.ds(c * CW * D, CW * D)], so)

        def wait_out(rbuf, so):
            pltpu.make_async_copy(rbuf, out_h.at[pl.ds(0, CW * D)],
                                  so).wait()

        fire(wid, t_a, sem_a)

        def body(t, _):
            c_a = wid + 2 * NW * t
            c_b = c_a + NW
            c_c = c_a + 2 * NW
            drain(c_a, t_a, sem_a)

            @pl.when(c_b < n_chunk)
            def _():
                fire(c_b, t_b, sem_b)

            @pl.when(t > 0)
            def _():
                wait_out(r_a, so_a)

            transpose(t_a, r_a)
            fire_out(c_a, r_a, so_a)

            @pl.when(c_b < n_chunk)
            def _():
                drain(c_b, t_b, sem_b)

                @pl.when(c_c < n_chunk)
                def _():
                    fire(c_c, t_a, sem_a)

                @pl.when(t > 0)
                def _():
                    wait_out(r_b, so_b)

                transpose(t_b, r_b)
                fire_out(c_b, r_b, so_b)

            return 0

        lax.fori_loop(0, n_pair, body, 0)
        wait_out(r_a, so_a)
        wait_out(r_b, so_b)

        if tailw:
            @pl.when(wid == NW - 1)
            def _():
                pltpu.sync_copy(tail_h, r_a.at[pl.ds(0, tailw * D)])
                pltpu.sync_copy(r_a.at[pl.ds(0, tailw * D)],
                                out_h.at[pl.ds(tail0 * D, tailw * D)])

    return stage(t2, tail_flat)


SB = 1


def _gather_pool_project(text, table2, W, b, S, B, V, D, O):
    bpw = B // NW
    assert S % (2 * SB) == 0

    @functools.partial(
        pl.kernel,
        out_type=jax.ShapeDtypeStruct((B * O,), jnp.float32),
        mesh=_sc_mesh(),
        compiler_params=pltpu.CompilerParams(
            needs_layout_passes=False, use_tc_tiling_on_sc=False),
        scratch_types=[
            pltpu.VMEM((S * bpw,), jnp.int32),
            pltpu.VMEM((SB * bpw, D), jnp.float32),
            pltpu.VMEM((SB * bpw, D), jnp.float32),
            pltpu.VMEM((SB * bpw, D), jnp.float32),
            pltpu.VMEM((SB * bpw, D), jnp.float32),
            pltpu.VMEM((bpw, D), jnp.float32),
            pltpu.SemaphoreType.DMA,
            pltpu.SemaphoreType.DMA,
            pltpu.SemaphoreType.DMA,
            pltpu.SemaphoreType.DMA,
            pltpu.VMEM((O, D), jnp.float32),
            pltpu.VMEM((L,), jnp.float32),
            pltpu.VMEM((bpw * D,), jnp.float32),
            pltpu.VMEM((bpw * O,), jnp.float32),
        ],
    )
    def fasttext_sc(text_h, table_h, w_h, b_h, out_h,
                    idx_v, rows0, rows1, rows2, rows3, acc_v,
                    sem0, sem1, sem2, sem3,
                    w_v, b_v, flat_v, out_v):
        wid = lax.axis_index("s") * NC + lax.axis_index("c")
        base = wid * bpw

        for s in range(S):
            pltpu.async_copy(text_h.at[s, pl.ds(base, bpw)],
                             idx_v.at[pl.ds(s * bpw, bpw)], sem0)
        for s in range(S):
            pltpu.make_async_copy(text_h.at[s, pl.ds(base, bpw)],
                                  idx_v.at[pl.ds(s * bpw, bpw)], sem0).wait()
        pltpu.sync_copy(w_h, w_v)
        pltpu.sync_copy(b_h, b_v.at[pl.ds(0, O)])

        zero = jnp.zeros((L,), jnp.float32)
        for r in range(bpw):
            for h in range(D // L):
                acc_v[r, pl.ds(h * L, L)] = zero

        def accumulate(buf):
            def acc16(i, _):
                r0 = i * 16
                for k in range(16):
                    for h in range(D // L):
                        plsc.addupdate(
                            acc_v.at[r0 + k, pl.ds(h * L, L)],
                            buf[r0 + k, pl.ds(h * L, L)])
                return 0
            lax.fori_loop(0, SB * bpw // 16, acc16, 0)

        nb = SB * bpw
        hb = nb // 2

        def gather_start(g, buf, sem):
            pltpu.async_copy(
                table_h.at[idx_v.at[pl.ds(g * nb, hb)]],
                buf.at[pl.ds(0, hb), :], sem)
            pltpu.async_copy(
                table_h.at[idx_v.at[pl.ds(g * nb + hb, hb)]],
                buf.at[pl.ds(hb, hb), :], sem)

        def gather_wait(g, buf, sem):
            pltpu.make_async_copy(
                table_h.at[idx_v.at[pl.ds(g * nb, hb)]],
                buf.at[pl.ds(0, hb), :], sem).wait()
            pltpu.make_async_copy(
                table_h.at[idx_v.at[pl.ds(g * nb + hb, hb)]],
                buf.at[pl.ds(hb, hb), :], sem).wait()

        ng = S // SB
        gather_start(0, rows0, sem0)

        def seq_pair(t, _):
            g = 2 * t
            gather_wait(g, rows0, sem0)
            gather_start(g + 1, rows1, sem1)
            accumulate(rows0)
            gather_wait(g + 1, rows1, sem1)

            @pl.when(g + 2 < ng)
            def _():
                gather_start(g + 2, rows0, sem0)

            accumulate(rows1)
            return 0

        lax.fori_loop(0, ng // 2, seq_pair, 0)

        for r in range(bpw):
            for h in range(D // L):
                flat_v[pl.ds(r * D + h * L, L)] = acc_v[r, pl.ds(h * L, L)]

        inv_s = jnp.float32(1.0 / S)
        lanes = lax.iota(jnp.int32, L)
        w_rows = [[w_v[o, pl.ds(h * L, L)] for h in range(D // L)]
                  for o in range(O)]
        ws = [[w_rows[o][d // L][d % L] for d in range(D)] for o in range(O)]
        b_vec = b_v[pl.ds(0, L)]
        bs = [b_vec[o] for o in range(O)]
        for g in range(bpw // L):
            row_idx = (g * L + lanes) * D
            outs = [jnp.zeros((L,), jnp.float32) for _ in range(O)]
            for d in range(D):
                vals = plsc.load_gather(flat_v, [row_idx + d])
                for o in range(O):
                    outs[o] = outs[o] + vals * ws[o][d]
            for o in range(O):
                res = outs[o] * inv_s + bs[o]
                plsc.store_scatter(out_v, [(g * L + lanes) * O + o], res)

        pltpu.sync_copy(out_v, out_h.at[pl.ds(base * O, bpw * O)])

    return fasttext_sc(text, table2, W, b)


@jax.jit
def kernel(text, table, W, b):
    S, B = text.shape
    V, D = table.shape
    O = W.shape[0]
    assert B % NW == 0 and D == 2 * L

    tail0 = (V // CW) * CW
    tail_flat = table[tail0:].reshape(-1)
    flat = _stage_row_major(table.T, tail_flat, V, D)
    table2 = flat.reshape(V, D)
    out = _gather_pool_project(text, table2, W, b, S, B, V, D, O)
    return out.reshape(B, O)


# speedup vs baseline: 1.4281x; 1.0097x over previous
import functools

import jax
import jax.numpy as jnp
from jax import lax
from jax.experimental import pallas as pl
from jax.experimental.pallas import tpu as pltpu
from jax.experimental.pallas import tpu_sc as plsc

NC = 2
NS = 16
L = 16
NW = NC * NS

CW = 512


def _sc_mesh():
    return plsc.VectorSubcoreMesh(
        core_axis_name="c", subcore_axis_name="s",
        num_cores=NC, num_subcores=NS)


def _stage_row_major(t2, tail_flat, V, D):
    n_chunk = V // CW
    tail0 = n_chunk * CW
    tailw = V - tail0
    n_pair = (n_chunk // NW + 1) // 2

    @functools.partial(
        pl.kernel,
        out_type=jax.ShapeDtypeStruct((V * D,), jnp.float32),
        mesh=_sc_mesh(),
        compiler_params=pltpu.CompilerParams(
            needs_layout_passes=False, use_tc_tiling_on_sc=True),
        scratch_types=[
            pltpu.VMEM((D, CW), jnp.float32),
            pltpu.VMEM((D, CW), jnp.float32),
            pltpu.VMEM((CW * D,), jnp.float32),
            pltpu.VMEM((CW * D,), jnp.float32),
            pltpu.SemaphoreType.DMA,
            pltpu.SemaphoreType.DMA,
            pltpu.SemaphoreType.DMA,
            pltpu.SemaphoreType.DMA,
        ],
    )
    def stage(t2_h, tail_h, out_h, t_a, t_b, r_a, r_b,
              sem_a, sem_b, so_a, so_b):
        wid = lax.axis_index("s") * NC + lax.axis_index("c")
        lanes = lax.iota(jnp.int32, L)
        row0 = lanes
        row1 = lanes + L

        def fire(c, tbuf, sem):
            pltpu.async_copy(t2_h.at[:, pl.ds(c * CW, CW)], tbuf, sem)

        def drain(c, tbuf, sem):
            pltpu.make_async_copy(t2_h.at[:, pl.ds(c * CW, CW)],
                                  tbuf, sem).wait()

        perm = [jnp.bitwise_and(lanes + j, L - 1) for j in range(L)]
        svec = [perm[j] * D + lanes for j in range(L)]
        rows_h = [row0, row1]

        def transpose(tbuf, rbuf):
            def tr(vb, _):
                gs = []
                for hb in range(D // L):
                    for j in range(L):
                        colv = perm[j] + L * vb
                        gs.append(plsc.load_gather(tbuf, [rows_h[hb], colv]))
                for hb in range(D // L):
                    for j in range(L):
                        sidx = svec[j] + (L * D * vb + L * hb)
                        plsc.store_scatter(rbuf, [sidx],
                                           gs[hb * L + j])
                return 0
            lax.fori_loop(0, CW // L, tr, 0)

        def fire_out(c, rbuf, so):
            pltpu.async_copy(rbuf, out_h.at[pl.ds(c * CW * D, CW * D)], so)

        def wait_out(rbuf, so):
            pltpu.make_async_copy(rbuf, out_h.at[pl.ds(0, CW * D)],
                                  so).wait()

        fire(wid, t_a, sem_a)

        def body(t, _):
            c_a = wid + 2 * NW * t
            c_b = c_a + NW
            c_c = c_a + 2 * NW
            drain(c_a, t_a, sem_a)

            @pl.when(c_b < n_chunk)
            def _():
                fire(c_b, t_b, sem_b)

            @pl.when(t > 0)
            def _():
                wait_out(r_a, so_a)

            transpose(t_a, r_a)
            fire_out(c_a, r_a, so_a)

            @pl.when(c_b < n_chunk)
            def _():
                drain(c_b, t_b, sem_b)

                @pl.when(c_c < n_chunk)
                def _():
                    fire(c_c, t_a, sem_a)

                @pl.when(t > 0)
                def _():
                    wait_out(r_b, so_b)

                transpose(t_b, r_b)
                fire_out(c_b, r_b, so_b)

            return 0

        lax.fori_loop(0, n_pair, body, 0)
        wait_out(r_a, so_a)
        wait_out(r_b, so_b)

        if tailw:
            @pl.when(wid == NW - 1)
            def _():
                pltpu.sync_copy(tail_h, r_a.at[pl.ds(0, tailw * D)])
                pltpu.sync_copy(r_a.at[pl.ds(0, tailw * D)],
                                out_h.at[pl.ds(tail0 * D, tailw * D)])

    return stage(t2, tail_flat)


SB = 1


def _gather_pool_project(text, table2, W, b, S, B, V, D, O):
    bpw = B // NW
    assert S % (2 * SB) == 0

    @functools.partial(
        pl.kernel,
        out_type=jax.ShapeDtypeStruct((B * O,), jnp.float32),
        mesh=_sc_mesh(),
        compiler_params=pltpu.CompilerParams(
            needs_layout_passes=False, use_tc_tiling_on_sc=False),
        scratch_types=[
            pltpu.VMEM((S, bpw), jnp.int32),
            pltpu.VMEM((SB * bpw, D), jnp.float32),
            pltpu.VMEM((SB * bpw, D), jnp.float32),
            pltpu.VMEM((bpw, D), jnp.float32),
            pltpu.SemaphoreType.DMA,
            pltpu.SemaphoreType.DMA,
            pltpu.VMEM((O, D), jnp.float32),
            pltpu.VMEM((L,), jnp.float32),
            pltpu.VMEM((bpw * D,), jnp.float32),
            pltpu.VMEM((bpw * O,), jnp.float32),
        ],
    )
    def fasttext_sc(text_h, table_h, w_h, b_h, out_h,
                    idx_v, rows0, rows1, acc_v, sem0, sem1,
                    w_v, b_v, flat_v, out_v):
        wid = lax.axis_index("s") * NC + lax.axis_index("c")
        base = wid * bpw

        pltpu.sync_copy(text_h.at[:, pl.ds(base, bpw)], idx_v)
        pltpu.sync_copy(w_h, w_v)
        pltpu.sync_copy(b_h, b_v.at[pl.ds(0, O)])

        zero = jnp.zeros((L,), jnp.float32)

        def z16(i, _):
            r0 = i * 16
            for k in range(16):
                for h in range(D // L):
                    acc_v[r0 + k, pl.ds(h * L, L)] = zero
            return 0

        lax.fori_loop(0, bpw // 16, z16, 0)

        def accumulate(buf):
            def acc16(i, _):
                r0 = i * 16
                for k in range(16):
                    for h in range(D // L):
                        plsc.addupdate(
                            acc_v.at[r0 + k, pl.ds(h * L, L)],
                            buf[r0 + k, pl.ds(h * L, L)])
                return 0
            lax.fori_loop(0, SB * bpw // 16, acc16, 0)

        hb = bpw // 2

        def gather_start(g, buf, sem):
            pltpu.async_copy(
                table_h.at[idx_v.at[g, pl.ds(0, hb)]],
                buf.at[pl.ds(0, hb), :], sem)
            pltpu.async_copy(
                table_h.at[idx_v.at[g, pl.ds(hb, hb)]],
                buf.at[pl.ds(hb, hb), :], sem)

        def gather_wait(g, buf, sem):
            pltpu.make_async_copy(
                table_h.at[idx_v.at[g, pl.ds(0, hb)]],
                buf.at[pl.ds(0, hb), :], sem).wait()
            pltpu.make_async_copy(
                table_h.at[idx_v.at[g, pl.ds(hb, hb)]],
                buf.at[pl.ds(hb, hb), :], sem).wait()

        ng = S // SB
        gather_start(0, rows0, sem0)

        def seq_pair(t, _):
            g = 2 * t
            gather_wait(g, rows0, sem0)
            gather_start(g + 1, rows1, sem1)
            accumulate(rows0)
            gather_wait(g + 1, rows1, sem1)

            @pl.when(g + 2 < ng)
            def _():
                gather_start(g + 2, rows0, sem0)

            accumulate(rows1)
            return 0

        lax.fori_loop(0, ng // 2, seq_pair, 0)

        def f16(i, _):
            r0 = i * 16
            for k in range(16):
                for h in range(D // L):
                    flat_v[pl.ds((r0 + k) * D + h * L, L)] = (
                        acc_v[r0 + k, pl.ds(h * L, L)])
            return 0

        lax.fori_loop(0, bpw // 16, f16, 0)

        inv_s = jnp.float32(1.0 / S)
        lanes = lax.iota(jnp.int32, L)
        w_rows = [[w_v[o, pl.ds(h * L, L)] for h in range(D // L)]
                  for o in range(O)]
        ws = [[w_rows[o][d // L][d % L] for d in range(D)] for o in range(O)]
        b_vec = b_v[pl.ds(0, L)]
        bs = [b_vec[o] for o in range(O)]
        for g in range(bpw // L):
            row_idx = (g * L + lanes) * D
            outs = [jnp.zeros((L,), jnp.float32) for _ in range(O)]
            for d in range(D):
                vals = plsc.load_gather(flat_v, [row_idx + d])
                for o in range(O):
                    outs[o] = outs[o] + vals * ws[o][d]
            for o in range(O):
                res = outs[o] * inv_s + bs[o]
                plsc.store_scatter(out_v, [(g * L + lanes) * O + o], res)

        pltpu.sync_copy(out_v, out_h.at[pl.ds(base * O, bpw * O)])

    return fasttext_sc(text, table2, W, b)


@jax.jit
def kernel(text, table, W, b):
    S, B = text.shape
    V, D = table.shape
    O = W.shape[0]
    assert B % NW == 0 and D == 2 * L

    tail0 = (V // CW) * CW
    tail_flat = table[tail0:].reshape(-1)
    flat = _stage_row_major(table.T, tail_flat, V, D)
    table2 = flat.reshape(V, D)
    out = _gather_pool_project(text, table2, W, b, S, B, V, D, O)
    return out.reshape(B, O)


# speedup vs baseline: 2.0633x; 1.4448x over previous
import functools

import jax
import jax.numpy as jnp
from jax import lax
from jax.experimental import pallas as pl
from jax.experimental.pallas import tpu as pltpu
from jax.experimental.pallas import tpu_sc as plsc

NC = 2
NS = 16
L = 16
NW = NC * NS

CW = 512


def _sc_mesh():
    return plsc.VectorSubcoreMesh(
        core_axis_name="c", subcore_axis_name="s",
        num_cores=NC, num_subcores=NS)


def _stage_row_major(t2, tail_flat, V, D):
    n_chunk = V // CW
    tail0 = n_chunk * CW
    tailw = V - tail0
    n_pair = (n_chunk // NW + 1) // 2

    @functools.partial(
        pl.kernel,
        out_type=jax.ShapeDtypeStruct((V * D,), jnp.float32),
        mesh=_sc_mesh(),
        compiler_params=pltpu.CompilerParams(
            needs_layout_passes=False, use_tc_tiling_on_sc=True),
        scratch_types=[
            pltpu.VMEM((D, CW), jnp.float32),
            pltpu.VMEM((D, CW), jnp.float32),
            pltpu.VMEM((CW * D,), jnp.float32),
            pltpu.VMEM((CW * D,), jnp.float32),
            pltpu.SemaphoreType.DMA,
            pltpu.SemaphoreType.DMA,
            pltpu.SemaphoreType.DMA,
            pltpu.SemaphoreType.DMA,
        ],
    )
    def stage(t2_h, tail_h, out_h, t_a, t_b, r_a, r_b,
              sem_a, sem_b, so_a, so_b):
        wid = lax.axis_index("s") * NC + lax.axis_index("c")
        lanes = lax.iota(jnp.int32, L)
        row0 = lanes
        row1 = lanes + L

        def fire(c, tbuf, sem):
            pltpu.async_copy(t2_h.at[:, pl.ds(c * CW, CW)], tbuf, sem)

        def drain(c, tbuf, sem):
            pltpu.make_async_copy(t2_h.at[:, pl.ds(c * CW, CW)],
                                  tbuf, sem).wait()

        perm = [jnp.bitwise_and(lanes + j, L - 1) for j in range(L)]
        svec = [perm[j] * D + lanes for j in range(L)]
        rows_h = [row0, row1]

        def transpose(tbuf, rbuf):
            def tr(vb, _):
                gs = []
                for hb in range(D // L):
                    for j in range(L):
                        colv = perm[j] + L * vb
                        gs.append(plsc.load_gather(tbuf, [rows_h[hb], colv]))
                for hb in range(D // L):
                    for j in range(L):
                        sidx = svec[j] + (L * D * vb + L * hb)
                        plsc.store_scatter(rbuf, [sidx],
                                           gs[hb * L + j])
                return 0
            lax.fori_loop(0, CW // L, tr, 0)

        def fire_out(c, rbuf, so):
            pltpu.async_copy(rbuf, out_h.at[pl.ds(c * CW * D, CW * D)], so)

        def wait_out(rbuf, so):
            pltpu.make_async_copy(rbuf, out_h.at[pl.ds(0, CW * D)],
                                  so).wait()

        fire(wid, t_a, sem_a)

        def body(t, _):
            c_a = wid + 2 * NW * t
            c_b = c_a + NW
            c_c = c_a + 2 * NW
            drain(c_a, t_a, sem_a)

            @pl.when(c_b < n_chunk)
            def _():
                fire(c_b, t_b, sem_b)

            @pl.when(t > 0)
            def _():
                wait_out(r_a, so_a)

            transpose(t_a, r_a)
            fire_out(c_a, r_a, so_a)

            @pl.when(c_b < n_chunk)
            def _():
                drain(c_b, t_b, sem_b)

                @pl.when(c_c < n_chunk)
                def _():
                    fire(c_c, t_a, sem_a)

                @pl.when(t > 0)
                def _():
                    wait_out(r_b, so_b)

                transpose(t_b, r_b)
                fire_out(c_b, r_b, so_b)

            return 0

        lax.fori_loop(0, n_pair, body, 0)
        wait_out(r_a, so_a)
        wait_out(r_b, so_b)

        if tailw:
            @pl.when(wid == NW - 1)
            def _():
                pltpu.sync_copy(tail_h, r_a.at[pl.ds(0, tailw * D)])
                pltpu.sync_copy(r_a.at[pl.ds(0, tailw * D)],
                                out_h.at[pl.ds(tail0 * D, tailw * D)])

    return stage(t2, tail_flat)


SB = 1


def _gather_pool_project(text, table2, W, b, S, B, V, D, O):
    bpw = B // NW
    assert S % (2 * SB) == 0

    @functools.partial(
        pl.kernel,
        out_type=jax.ShapeDtypeStruct((B * O,), jnp.float32),
        mesh=_sc_mesh(),
        compiler_params=pltpu.CompilerParams(
            needs_layout_passes=False, use_tc_tiling_on_sc=False),
        scratch_types=[
            pltpu.VMEM((S, bpw), jnp.int32),
            pltpu.VMEM((SB * bpw, D), jnp.float32),
            pltpu.VMEM((SB * bpw, D), jnp.float32),
            pltpu.VMEM((SB * bpw, D), jnp.float32),
            pltpu.VMEM((SB * bpw, D), jnp.float32),
            pltpu.VMEM((bpw, D), jnp.float32),
            pltpu.SemaphoreType.DMA,
            pltpu.SemaphoreType.DMA,
            pltpu.SemaphoreType.DMA,
            pltpu.SemaphoreType.DMA,
            pltpu.VMEM((O, D), jnp.float32),
            pltpu.VMEM((L,), jnp.float32),
            pltpu.VMEM((bpw * D,), jnp.float32),
            pltpu.VMEM((bpw * O,), jnp.float32),
        ],
    )
    def fasttext_sc(text_h, table_h, w_h, b_h, out_h,
                    idx_v, rows0, rows1, rows2, rows3, acc_v,
                    sem0, sem1, sem2, sem3,
                    w_v, b_v, flat_v, out_v):
        wid = lax.axis_index("s") * NC + lax.axis_index("c")
        base = wid * bpw

        pltpu.sync_copy(text_h.at[:, pl.ds(base, bpw)], idx_v)
        pltpu.sync_copy(w_h, w_v)
        pltpu.sync_copy(b_h, b_v.at[pl.ds(0, O)])

        zero = jnp.zeros((L,), jnp.float32)

        def z16(i, _):
            r0 = i * 16
            for k in range(16):
                for h in range(D // L):
                    acc_v[r0 + k, pl.ds(h * L, L)] = zero
            return 0

        lax.fori_loop(0, bpw // 16, z16, 0)

        def accumulate(buf):
            def acc16(i, _):
                r0 = i * 16
                for k in range(16):
                    for h in range(D // L):
                        plsc.addupdate(
                            acc_v.at[r0 + k, pl.ds(h * L, L)],
                            buf[r0 + k, pl.ds(h * L, L)])
                return 0
            lax.fori_loop(0, SB * bpw // 16, acc16, 0)

        hb = bpw // 2

        def gather_start(g, buf, sem):
            pltpu.async_copy(
                table_h.at[idx_v.at[g, pl.ds(0, hb)]],
                buf.at[pl.ds(0, hb), :], sem)
            pltpu.async_copy(
                table_h.at[idx_v.at[g, pl.ds(hb, hb)]],
                buf.at[pl.ds(hb, hb), :], sem)

        def gather_wait(g, buf, sem):
            pltpu.make_async_copy(
                table_h.at[idx_v.at[g, pl.ds(0, hb)]],
                buf.at[pl.ds(0, hb), :], sem).wait()
            pltpu.make_async_copy(
                table_h.at[idx_v.at[g, pl.ds(hb, hb)]],
                buf.at[pl.ds(hb, hb), :], sem).wait()

        ng = S // SB
        ring = [(rows0, sem0), (rows1, sem1), (rows2, sem2), (rows3, sem3)]
        nr = len(ring)
        for k in range(nr):
            gather_start(k, *ring[k])

        def seq_block(t, _):
            g0 = nr * t
            for k in range(nr):
                buf, sem = ring[k]
                gather_wait(g0 + k, buf, sem)
                accumulate(buf)

                @pl.when(g0 + k + nr < ng)
                def _():
                    gather_start(g0 + k + nr, buf, sem)

            return 0

        lax.fori_loop(0, ng // nr, seq_block, 0)

        def f16(i, _):
            r0 = i * 16
            for k in range(16):
                for h in range(D // L):
                    flat_v[pl.ds((r0 + k) * D + h * L, L)] = (
                        acc_v[r0 + k, pl.ds(h * L, L)])
            return 0

        lax.fori_loop(0, bpw // 16, f16, 0)

        inv_s = jnp.float32(1.0 / S)
        lanes = lax.iota(jnp.int32, L)
        w_rows = [[w_v[o, pl.ds(h * L, L)] for h in range(D // L)]
                  for o in range(O)]
        ws = [[w_rows[o][d // L][d % L] for d in range(D)] for o in range(O)]
        b_vec = b_v[pl.ds(0, L)]
        bs = [b_vec[o] for o in range(O)]
        for g in range(bpw // L):
            row_idx = (g * L + lanes) * D
            outs = [jnp.zeros((L,), jnp.float32) for _ in range(O)]
            for d in range(D):
                vals = plsc.load_gather(flat_v, [row_idx + d])
                for o in range(O):
                    outs[o] = outs[o] + vals * ws[o][d]
            for o in range(O):
                res = outs[o] * inv_s + bs[o]
                plsc.store_scatter(out_v, [(g * L + lanes) * O + o], res)

        pltpu.sync_copy(out_v, out_h.at[pl.ds(base * O, bpw * O)])

    return fasttext_sc(text, table2, W, b)


@jax.jit
def kernel(text, table, W, b):
    S, B = text.shape
    V, D = table.shape
    O = W.shape[0]
    assert B % NW == 0 and D == 2 * L

    tail0 = (V // CW) * CW
    tail_flat = table[tail0:].reshape(-1)
    flat = _stage_row_major(table.T, tail_flat, V, D)
    table2 = flat.reshape(V, D)
    out = _gather_pool_project(text, table2, W, b, S, B, V, D, O)
    return out.reshape(B, O)
